# baseline (speedup 1.0000x reference)
import functools

import jax
import jax.numpy as jnp
from jax import lax
from jax.experimental import pallas as pl
from jax.experimental.pallas import tpu as pltpu
from jax.experimental.pallas import tpu_sc as plsc

_N = 10000
_E = 320000
_D = 128
_H = 25
_HP = 32
_NPAD = 10240
_NP4 = _NPAD // 4
_NV = _N // 4
_NW = 32
_C = 128
_CH = 80
_EPAD = _NW * _CH * _C
_RPS = _NPAD // 16
_NB = 8
_ZC = 128

_f32 = jnp.float32
_sc_mesh = plsc.VectorSubcoreMesh(core_axis_name="c", subcore_axis_name="s")
_sc_params = pltpu.CompilerParams(use_tc_tiling_on_sc=False)


@functools.partial(
    pl.kernel,
    out_type=jax.ShapeDtypeStruct((2, _NPAD), _f32),
    mesh=_sc_mesh,
    scratch_types=[
        pltpu.VMEM((_CH, _C), jnp.int32),
        pltpu.VMEM((_C,), _f32),
        pltpu.VMEM((_RPS,), _f32),
        pltpu.VMEM_SHARED((_NPAD,), _f32),
        pltpu.SemaphoreType.DMA,
    ],
    compiler_params=_sc_params,
)
def _sc_deg(epad_hbm, out_hbm, dst_v, ones_v, zbuf, deg_sh, dsem):
    c = lax.axis_index("c")
    s = lax.axis_index("s")
    wid = c * 16 + s

    def _fill_zero(i, _):
        zbuf[pl.ds(i * 16, 16)] = jnp.zeros((16,), _f32)
        return 0

    lax.fori_loop(0, _RPS // 16, _fill_zero, 0)

    def _fill_one(i, _):
        ones_v[pl.ds(i * 16, 16)] = jnp.ones((16,), _f32)
        return 0

    lax.fori_loop(0, _C // 16, _fill_one, 0)

    pltpu.sync_copy(zbuf, deg_sh.at[pl.ds(s * _RPS, _RPS)])
    pltpu.sync_copy(epad_hbm.at[1, wid], dst_v)
    plsc.subcore_barrier()

    def _group(i, _):
        for b in range(8):
            pltpu.async_copy(ones_v, deg_sh.at[dst_v.at[i * 8 + b]], dsem,
                             add=True)
        for b in range(8):
            pltpu.make_async_copy(ones_v, deg_sh.at[dst_v.at[i * 8 + b]],
                                  dsem).wait()
        return 0

    lax.fori_loop(0, _CH // 8, _group, 0)
    plsc.subcore_barrier()
    pltpu.sync_copy(deg_sh.at[pl.ds(s * _RPS, _RPS)],
                    out_hbm.at[c, pl.ds(s * _RPS, _RPS)])


@functools.partial(
    pl.kernel,
    out_type=jax.ShapeDtypeStruct((2, _NPAD, _HP), _f32),
    mesh=_sc_mesh,
    scratch_types=[
        pltpu.VMEM((_CH, _C), jnp.int32),
        pltpu.VMEM((_CH, _C), jnp.int32),
        pltpu.VMEM((_NB, _C, _HP), _f32),
        pltpu.VMEM((_ZC, _HP), _f32),
        pltpu.VMEM_SHARED((_NPAD, _HP), _f32),
        pltpu.VMEM_SHARED((_NPAD, _HP), _f32),
        pltpu.SemaphoreType.DMA((_NB,)),
        pltpu.SemaphoreType.DMA((_NB,)),
        pltpu.SemaphoreType.DMA,
        pltpu.SemaphoreType.DMA,
    ],
    compiler_params=_sc_params,
)
def _sc_agg(hws_hbm, epad_hbm, out_hbm,
            src_v, dst_v, rows_v, zbuf, acc_sh, hws_sh, gsem, ssem, zsem, hsem):
    c = lax.axis_index("c")
    s = lax.axis_index("s")
    wid = c * 16 + s

    def _fill_zero(i, _):
        zbuf[i, pl.ds(0, 16)] = jnp.zeros((16,), _f32)
        zbuf[i, pl.ds(16, 16)] = jnp.zeros((16,), _f32)
        return 0

    lax.fori_loop(0, _ZC, _fill_zero, 0)

    pltpu.async_copy(hws_hbm.at[pl.ds(s * _RPS, _RPS)],
                     hws_sh.at[pl.ds(s * _RPS, _RPS)], hsem)
    for k in range(_RPS // _ZC):
        pltpu.async_copy(zbuf, acc_sh.at[pl.ds(s * _RPS + k * _ZC, _ZC)], zsem)
    pltpu.sync_copy(epad_hbm.at[0, wid], src_v)
    pltpu.sync_copy(epad_hbm.at[1, wid], dst_v)
    for k in range(_RPS // _ZC):
        pltpu.make_async_copy(
            zbuf, acc_sh.at[pl.ds(s * _RPS + k * _ZC, _ZC)], zsem).wait()
    pltpu.make_async_copy(hws_hbm.at[pl.ds(s * _RPS, _RPS)],
                          hws_sh.at[pl.ds(s * _RPS, _RPS)], hsem).wait()
    plsc.subcore_barrier()

    for b in range(_NB):
        pltpu.async_copy(hws_sh.at[src_v.at[b]], rows_v.at[b], gsem.at[b])

    def _group(i, _):
        base = i * _NB
        for b in range(_NB):
            j = base + b
            pltpu.make_async_copy(hws_sh.at[src_v.at[j]], rows_v.at[b],
                                  gsem.at[b]).wait()
            pltpu.async_copy(rows_v.at[b], acc_sh.at[dst_v.at[j]], ssem.at[b],
                             add=True)
        for b in range(_NB):
            j = base + b
            pltpu.make_async_copy(rows_v.at[b], acc_sh.at[dst_v.at[j]],
                                  ssem.at[b]).wait()
            jn = lax.min(j + _NB, _CH - 1)
            pltpu.async_copy(hws_sh.at[src_v.at[jn]], rows_v.at[b],
                             gsem.at[b])
        return 0

    lax.fori_loop(0, _CH // _NB, _group, 0)
    for b in range(_NB):
        pltpu.make_async_copy(hws_sh.at[src_v.at[0]], rows_v.at[b],
                              gsem.at[b]).wait()
    plsc.subcore_barrier()
    pltpu.sync_copy(acc_sh.at[pl.ds(s * _RPS, _RPS)],
                    out_hbm.at[c, pl.ds(s * _RPS, _RPS)])


def _prep_body(deg_ref, xp_ref, w1bd_ref, dinv_ref, hws_ref):
    deg = deg_ref[0] + deg_ref[1] + 1.0
    dinv4 = lax.rsqrt(deg)
    rows = lax.broadcasted_iota(jnp.int32, (_NP4, 4), 0)
    dinv4 = jnp.where(rows < _NV, dinv4, 0.0)
    dinv = jnp.concatenate(
        [jnp.broadcast_to(dinv4[:, k:k + 1], (_NP4, _HP)) for k in range(4)],
        axis=1)
    dinv_ref[...] = dinv
    hws_ref[...] = jnp.dot(xp_ref[...], w1bd_ref[...],
                           preferred_element_type=_f32) * dinv


def _mid_body(acc_ref, hws_ref, dinv_ref, b_ref, wbd_ref, out_ref):
    dinv = dinv_ref[...]
    t = (acc_ref[0] + acc_ref[1] + hws_ref[...]) * dinv + b_ref[...]
    h = jnp.maximum(t, 0.0)
    out_ref[...] = jnp.dot(h, wbd_ref[...],
                           preferred_element_type=_f32) * dinv


def _fin_body(acc_ref, hws_ref, dinv_ref, b_ref, wcbd_ref, bc_ref, out_ref):
    t = (acc_ref[0] + acc_ref[1] + hws_ref[...]) * dinv_ref[...] + b_ref[...]
    h = jnp.maximum(t, 0.0)
    out_ref[...] = jnp.dot(h, wcbd_ref[...],
                           preferred_element_type=_f32) + bc_ref[...]


_tc_prep = pl.pallas_call(
    _prep_body,
    out_shape=(jax.ShapeDtypeStruct((_NP4, 128), _f32),
               jax.ShapeDtypeStruct((_NP4, 128), _f32)),
)

_tc_mid = pl.pallas_call(
    _mid_body,
    out_shape=jax.ShapeDtypeStruct((_NP4, 128), _f32),
)

_tc_fin = pl.pallas_call(
    _fin_body,
    out_shape=jax.ShapeDtypeStruct((_NP4, 4 * _D), _f32),
)


def kernel(x, edge_index, w1, b1, w2, b2, w3, b3, w4, b4, w5, b5, wc, bc):
    eye4 = jnp.eye(4, dtype=_f32)

    def bd(w):
        return jnp.kron(eye4, w)

    def padw(w):
        return jnp.pad(w, ((0, _HP - _H), (0, _HP - _H)))

    def padb(b):
        return jnp.tile(jnp.pad(b, (0, _HP - _H)), 4).reshape(1, 128)

    fill = (jnp.arange(_EPAD - _E, dtype=jnp.int32) % 64) + _N
    epad = jnp.concatenate(
        [edge_index, jnp.stack([fill, fill])], axis=1).reshape(2, _NW, _CH, _C)

    deg2 = _sc_deg(epad)
    deg_r = deg2.reshape(2, _NP4, 4)

    x_p = jnp.pad(x, ((0, _NPAD - _N), (0, 0))).reshape(_NP4, 4 * _D)
    w1bd = bd(jnp.pad(w1, ((0, 0), (0, _HP - _H))))
    dinv, hws = _tc_prep(deg_r, x_p, w1bd)

    mids = [(padb(b1), bd(padw(w2))), (padb(b2), bd(padw(w3))),
            (padb(b3), bd(padw(w4))), (padb(b4), bd(padw(w5)))]
    for b_p, w_p in mids:
        acc = _sc_agg(hws.reshape(_NPAD, _HP), epad)
        hws = _tc_mid(acc.reshape(2, _NP4, 128), hws, dinv, b_p, w_p)

    acc = _sc_agg(hws.reshape(_NPAD, _HP), epad)
    wcbd = bd(jnp.pad(wc, ((0, _HP - _H), (0, 0))))
    bc_p = jnp.tile(bc, 4).reshape(1, 4 * _D)
    out = _tc_fin(acc.reshape(2, _NP4, 128), hws, dinv, padb(b5), wcbd, bc_p)
    return out.reshape(_NPAD, _D)[:_N]


# speedup vs baseline: 1.2383x; 1.2383x over previous
import functools

import jax
import jax.numpy as jnp
from jax import lax
from jax.experimental import pallas as pl
from jax.experimental.pallas import tpu as pltpu
from jax.experimental.pallas import tpu_sc as plsc

_N = 10000
_E = 320000
_D = 128
_H = 25
_HP = 32
_NPAD = 10240
_NP4 = _NPAD // 4
_NV = _N // 4
_NW = 32
_C = 128
_CH = 80
_EPAD = _NW * _CH * _C
_RPS = _NPAD // 16
_ER = _E // _C
_CHR = _ER - (_NW - 1) * _CH
_CHF = _CH - _CHR
_NB = 8

_f32 = jnp.float32
_sc_mesh = plsc.VectorSubcoreMesh(core_axis_name="c", subcore_axis_name="s")
_sc_params = pltpu.CompilerParams(use_tc_tiling_on_sc=False)


@functools.partial(
    pl.kernel,
    out_type=jax.ShapeDtypeStruct((2, _NPAD), _f32),
    mesh=_sc_mesh,
    scratch_types=[
        pltpu.VMEM((_CH, _C), jnp.int32),
        pltpu.VMEM((_C,), _f32),
        pltpu.VMEM((_RPS,), _f32),
        pltpu.VMEM_SHARED((_NPAD,), _f32),
        pltpu.SemaphoreType.DMA,
    ],
    compiler_params=_sc_params,
)
def _sc_deg(er_hbm, fill_hbm, out_hbm, dst_v, ones_v, zbuf, deg_sh, dsem):
    c = lax.axis_index("c")
    s = lax.axis_index("s")
    wid = c * 16 + s

    def _fill_zero(i, _):
        zbuf[pl.ds(i * 16, 16)] = jnp.zeros((16,), _f32)
        return 0

    lax.fori_loop(0, _RPS // 16, _fill_zero, 0)

    def _fill_one(i, _):
        ones_v[pl.ds(i * 16, 16)] = jnp.ones((16,), _f32)
        return 0

    lax.fori_loop(0, _C // 16, _fill_one, 0)

    pltpu.sync_copy(zbuf, deg_sh.at[pl.ds(s * _RPS, _RPS)])

    @pl.when(wid < _NW - 1)
    def _():
        pltpu.sync_copy(er_hbm.at[1, pl.ds(wid * _CH, _CH)], dst_v)

    @pl.when(wid == _NW - 1)
    def _():
        pltpu.sync_copy(er_hbm.at[1, pl.ds((_NW - 1) * _CH, _CHR)],
                        dst_v.at[pl.ds(0, _CHR)])
        pltpu.sync_copy(fill_hbm.at[1], dst_v.at[pl.ds(_CHR, _CHF)])

    plsc.subcore_barrier()

    def _group(i, _):
        for b in range(8):
            pltpu.async_copy(ones_v, deg_sh.at[dst_v.at[i * 8 + b]], dsem,
                             add=True)
        for b in range(8):
            pltpu.make_async_copy(ones_v, deg_sh.at[dst_v.at[i * 8 + b]],
                                  dsem).wait()
        return 0

    lax.fori_loop(0, _CH // 8, _group, 0)
    plsc.subcore_barrier()
    pltpu.sync_copy(deg_sh.at[pl.ds(s * _RPS, _RPS)],
                    out_hbm.at[c, pl.ds(s * _RPS, _RPS)])


@functools.partial(
    pl.kernel,
    out_type=jax.ShapeDtypeStruct((2, _NPAD, _HP), _f32),
    mesh=_sc_mesh,
    scratch_types=[
        pltpu.VMEM((_CH, _C), jnp.int32),
        pltpu.VMEM((_CH, _C), jnp.int32),
        pltpu.VMEM((_NB, _C, _HP), _f32),
        pltpu.VMEM((_C, _HP), _f32),
        pltpu.VMEM_SHARED((_NPAD, _HP), _f32),
        pltpu.SemaphoreType.DMA((_NB,)),
        pltpu.SemaphoreType.DMA((_NB,)),
        pltpu.SemaphoreType.DMA,
    ],
    compiler_params=_sc_params,
)
def _sc_agg(hws_hbm, er_hbm, fill_hbm, out_hbm,
            src_v, dst_v, rows_v, zbuf, acc_sh, gsem, ssem, zsem):
    c = lax.axis_index("c")
    s = lax.axis_index("s")
    wid = c * 16 + s

    def _fill_zero(i, _):
        zbuf[i, pl.ds(0, 16)] = jnp.zeros((16,), _f32)
        zbuf[i, pl.ds(16, 16)] = jnp.zeros((16,), _f32)
        return 0

    lax.fori_loop(0, _C, _fill_zero, 0)

    for k in range(_RPS // _C):
        pltpu.async_copy(zbuf, acc_sh.at[pl.ds(s * _RPS + k * _C, _C)], zsem)
    @pl.when(wid < _NW - 1)
    def _():
        pltpu.sync_copy(er_hbm.at[0, pl.ds(wid * _CH, _CH)], src_v)
        pltpu.sync_copy(er_hbm.at[1, pl.ds(wid * _CH, _CH)], dst_v)

    @pl.when(wid == _NW - 1)
    def _():
        pltpu.sync_copy(er_hbm.at[0, pl.ds((_NW - 1) * _CH, _CHR)],
                        src_v.at[pl.ds(0, _CHR)])
        pltpu.sync_copy(fill_hbm.at[0], src_v.at[pl.ds(_CHR, _CHF)])
        pltpu.sync_copy(er_hbm.at[1, pl.ds((_NW - 1) * _CH, _CHR)],
                        dst_v.at[pl.ds(0, _CHR)])
        pltpu.sync_copy(fill_hbm.at[1], dst_v.at[pl.ds(_CHR, _CHF)])
    for k in range(_RPS // _C):
        pltpu.make_async_copy(
            zbuf, acc_sh.at[pl.ds(s * _RPS + k * _C, _C)], zsem).wait()
    plsc.subcore_barrier()

    for b in range(_NB):
        pltpu.async_copy(hws_hbm.at[src_v.at[b]], rows_v.at[b], gsem.at[b])

    def _group(i, _):
        base = i * _NB
        for b in range(_NB):
            j = base + b
            pltpu.make_async_copy(hws_hbm.at[src_v.at[j]], rows_v.at[b],
                                  gsem.at[b]).wait()
            pltpu.async_copy(rows_v.at[b], acc_sh.at[dst_v.at[j]], ssem.at[b],
                             add=True)
        for b in range(_NB):
            j = base + b
            pltpu.make_async_copy(rows_v.at[b], acc_sh.at[dst_v.at[j]],
                                  ssem.at[b]).wait()
            jn = lax.min(j + _NB, _CH - 1)
            pltpu.async_copy(hws_hbm.at[src_v.at[jn]], rows_v.at[b],
                             gsem.at[b])
        return 0

    lax.fori_loop(0, _CH // _NB, _group, 0)
    for b in range(_NB):
        pltpu.make_async_copy(hws_hbm.at[src_v.at[0]], rows_v.at[b],
                              gsem.at[b]).wait()
    plsc.subcore_barrier()
    pltpu.sync_copy(acc_sh.at[pl.ds(s * _RPS, _RPS)],
                    out_hbm.at[c, pl.ds(s * _RPS, _RPS)])


def _prep_body(deg_ref, x_ref, w1_ref, dinv_ref, hws_ref):
    deg = deg_ref[0] + deg_ref[1] + 1.0
    dinv4 = lax.rsqrt(deg)
    rows = lax.broadcasted_iota(jnp.int32, (_NP4, 4), 0)
    dinv4 = jnp.where(rows < _NV, dinv4, 0.0)
    dinv = jnp.concatenate(
        [jnp.broadcast_to(dinv4[:, k:k + 1], (_NP4, _HP)) for k in range(4)],
        axis=1)
    dinv_ref[...] = dinv
    hws_ref[...] = jnp.dot(x_ref[...], w1_ref[...],
                           preferred_element_type=_f32) * dinv


def _mid_body(acc_ref, hws_ref, dinv_ref, b_ref, wbd_ref, out_ref):
    dinv = dinv_ref[...]
    t = (acc_ref[0] + acc_ref[1] + hws_ref[...]) * dinv + b_ref[...]
    h = jnp.maximum(t, 0.0)
    out_ref[...] = jnp.dot(h, wbd_ref[...],
                           preferred_element_type=_f32) * dinv


def _fin_body(acc_ref, hws_ref, dinv_ref, b_ref, wc_ref, bc_ref, out_ref):
    t = (acc_ref[0] + acc_ref[1] + hws_ref[...]) * dinv_ref[...] + b_ref[...]
    h = jnp.maximum(t, 0.0)
    out_ref[...] = jnp.dot(h, wc_ref[...],
                           preferred_element_type=_f32) + bc_ref[...]


_tc_prep = pl.pallas_call(
    _prep_body,
    out_shape=(jax.ShapeDtypeStruct((_NP4, 128), _f32),
               jax.ShapeDtypeStruct((_NP4, 128), _f32)),
)

_tc_mid = pl.pallas_call(
    _mid_body,
    out_shape=jax.ShapeDtypeStruct((_NP4, 128), _f32),
)

_tc_fin = pl.pallas_call(
    _fin_body,
    out_shape=jax.ShapeDtypeStruct((_NP4, 4 * _D), _f32),
)


def kernel(x, edge_index, w1, b1, w2, b2, w3, b3, w4, b4, w5, b5, wc, bc):
    eye4 = jnp.eye(4, dtype=_f32)

    def bd(w):
        return jnp.kron(eye4, w)

    def padw(w):
        return jnp.pad(w, ((0, _HP - _H), (0, _HP - _H)))

    def padb(b):
        return jnp.tile(jnp.pad(b, (0, _HP - _H)), 4).reshape(1, 128)

    er = edge_index.reshape(2, _ER, _C)
    fillv = (jnp.arange(_CHF * _C, dtype=jnp.int32) % 64) + _N
    fill2 = jnp.stack([fillv, fillv]).reshape(2, _CHF, _C)

    deg2 = _sc_deg(er, fill2)
    deg_r = deg2.reshape(2, _NP4, 4)

    x_p = jnp.pad(x, ((0, _NPAD - _N), (0, 0))).reshape(_NP4, 4 * _D)
    w1bd = bd(jnp.pad(w1, ((0, 0), (0, _HP - _H))))
    dinv, hws = _tc_prep(deg_r, x_p, w1bd)

    mids = [(padb(b1), bd(padw(w2))), (padb(b2), bd(padw(w3))),
            (padb(b3), bd(padw(w4))), (padb(b4), bd(padw(w5)))]
    for b_p, w_p in mids:
        acc = _sc_agg(hws.reshape(_NPAD, _HP), er, fill2)
        hws = _tc_mid(acc.reshape(2, _NP4, 128), hws, dinv, b_p, w_p)

    acc = _sc_agg(hws.reshape(_NPAD, _HP), er, fill2)
    wcbd = bd(jnp.pad(wc, ((0, _HP - _H), (0, 0))))
    out = _tc_fin(acc.reshape(2, _NP4, 128), hws, dinv, padb(b5), wcbd,
                  jnp.tile(bc, 4).reshape(1, 4 * _D))
    return out.reshape(_NPAD, _D)[:_N]
